# Initial kernel scaffold; baseline (speedup 1.0000x reference)
import jax
import jax.numpy as jnp
from jax.experimental import pallas as pl


def kernel(feat_list, W, b, caps_basis):
    raise NotImplementedError("write your pallas kernel here")


# speedup vs baseline: 1.0048x; 1.0048x over previous
import jax
import jax.numpy as jnp
from jax.experimental import pallas as pl
from jax.experimental.pallas import tpu as pltpu

_DEPTH = 384
_RT = 512


def _stats_body(x_ref, w_ref, bias_ref, basis_ref, map_ref, max_ref, den_ref):
    r = pl.program_id(0)
    b = pl.program_id(1)
    base = r * _RT

    x = x_ref[b, pl.ds(base, _RT)]
    feat = jnp.dot(x, w_ref[...], preferred_element_type=jnp.float32)
    feat = feat + bias_ref[...]
    m = feat * basis_ref[...]
    map_ref[0] = m

    tmax = jnp.max(m, axis=0, keepdims=True)
    tsum = jnp.sum(jnp.exp(m - tmax), axis=0, keepdims=True)

    @pl.when(r == 0)
    def _():
        max_ref[pl.ds(b, 1), :] = tmax
        den_ref[pl.ds(b, 1), :] = tsum

    @pl.when(r > 0)
    def _():
        old_max = max_ref[pl.ds(b, 1), :]
        new_max = jnp.maximum(old_max, tmax)
        den_ref[pl.ds(b, 1), :] = (den_ref[pl.ds(b, 1), :]
                                   * jnp.exp(old_max - new_max)
                                   + tsum * jnp.exp(tmax - new_max))
        max_ref[pl.ds(b, 1), :] = new_max


def _out_body(x_ref, w_ref, bias_ref, basis_ref, max_ref, den_ref,
              out_ref, found_ref):
    r = pl.program_id(0)
    b = pl.program_id(1)
    base = r * _RT
    cout = basis_ref.shape[1]
    num_caps = x_ref.shape[1]

    x = x_ref[b, pl.ds(base, _RT)]
    feat = jnp.dot(x, w_ref[...], preferred_element_type=jnp.float32)
    feat = feat + bias_ref[...]
    m = feat * basis_ref[...]

    mx = max_ref[pl.ds(b, 1), :]
    den = den_ref[pl.ds(b, 1), :]
    a = jnp.exp(m - mx) / den
    amax = 1.0 / den

    tie = a == amax
    tie_f = tie.astype(jnp.float32)
    rows = jax.lax.broadcasted_iota(jnp.int32, m.shape, 0)
    first = jnp.argmax(tie_f, axis=0).astype(jnp.int32).reshape(1, -1)
    have = jnp.max(tie_f, axis=0, keepdims=True)

    prev = jnp.where(r > 0, found_ref[pl.ds(b, 1), :], 0)
    own = (have > 0.0) & (prev == 0)
    found_ref[pl.ds(b, 1), :] = prev | have.astype(jnp.int32)
    sel = tie & (rows == first) & own

    sel_paired = jnp.concatenate(
        [sel[:, :_DEPTH], sel[:, :_DEPTH], sel[:, 2 * _DEPTH:]], axis=1)

    av = a * feat
    cols = jax.lax.broadcasted_iota(jnp.int32, m.shape, 1)
    out_ref[0] = jnp.where(cols >= 2 * _DEPTH, av * (1.0 / num_caps),
                           jnp.where(sel_paired, av, 0.0))


def kernel(feat_list, W, b, caps_basis):
    x = feat_list[-2]
    Bv, Nv = x.shape[0], x.shape[1]
    cin = x.shape[-1]
    num_caps = caps_basis.shape[1]
    cout = caps_basis.shape[3]
    x = x.reshape(Bv, Nv * Nv, cin)
    basis = caps_basis.reshape(num_caps, cout)
    bias2 = b.reshape(1, cout)
    n_r = num_caps // _RT
    f32 = jnp.float32

    attn_map, mx, den = pl.pallas_call(
        _stats_body,
        grid=(n_r, Bv),
        in_specs=[
            pl.BlockSpec((Bv, num_caps, cin), lambda r, bb: (0, 0, 0)),
            pl.BlockSpec((cin, cout), lambda r, bb: (0, 0)),
            pl.BlockSpec((1, cout), lambda r, bb: (0, 0)),
            pl.BlockSpec((_RT, cout), lambda r, bb: (r, 0)),
        ],
        out_specs=(
            pl.BlockSpec((1, _RT, cout), lambda r, bb: (bb, r, 0)),
            pl.BlockSpec((Bv, cout), lambda r, bb: (0, 0)),
            pl.BlockSpec((Bv, cout), lambda r, bb: (0, 0)),
        ),
        out_shape=(
            jax.ShapeDtypeStruct((Bv, num_caps, cout), f32),
            jax.ShapeDtypeStruct((Bv, cout), f32),
            jax.ShapeDtypeStruct((Bv, cout), f32),
        ),
        compiler_params=pltpu.CompilerParams(
            dimension_semantics=("arbitrary", "arbitrary"),
        ),
    )(x, W, bias2, basis)

    out = pl.pallas_call(
        _out_body,
        grid=(n_r, Bv),
        in_specs=[
            pl.BlockSpec((Bv, num_caps, cin), lambda r, bb: (0, 0, 0)),
            pl.BlockSpec((cin, cout), lambda r, bb: (0, 0)),
            pl.BlockSpec((1, cout), lambda r, bb: (0, 0)),
            pl.BlockSpec((_RT, cout), lambda r, bb: (r, 0)),
            pl.BlockSpec((Bv, cout), lambda r, bb: (0, 0)),
            pl.BlockSpec((Bv, cout), lambda r, bb: (0, 0)),
        ],
        out_specs=pl.BlockSpec((1, _RT, cout), lambda r, bb: (bb, r, 0)),
        out_shape=jax.ShapeDtypeStruct((Bv, num_caps, cout), f32),
        scratch_shapes=[pltpu.VMEM((Bv, cout), jnp.int32)],
        compiler_params=pltpu.CompilerParams(
            dimension_semantics=("arbitrary", "arbitrary"),
        ),
    )(x, W, bias2, basis, mx, den)
    return (out, attn_map)


# speedup vs baseline: 1.0800x; 1.0749x over previous
import jax
import jax.numpy as jnp
from jax.experimental import pallas as pl
from jax.experimental.pallas import tpu as pltpu

_DEPTH = 384
_RT = 1024


def _stats_body(x_ref, w_ref, bias_ref, basis_ref, map_ref, max_ref, den_ref):
    r = pl.program_id(0)
    b = pl.program_id(1)
    base = r * _RT

    x = x_ref[0, b, pl.ds(base, _RT)]
    feat = jnp.dot(x, w_ref[...], preferred_element_type=jnp.float32)
    feat = feat + bias_ref[...]
    m = feat * basis_ref[...]
    map_ref[0] = m

    tmax = jnp.max(m, axis=0, keepdims=True)
    tsum = jnp.sum(jnp.exp(m - tmax), axis=0, keepdims=True)

    @pl.when(r == 0)
    def _():
        max_ref[pl.ds(b, 1), :] = tmax
        den_ref[pl.ds(b, 1), :] = tsum

    @pl.when(r > 0)
    def _():
        old_max = max_ref[pl.ds(b, 1), :]
        new_max = jnp.maximum(old_max, tmax)
        den_ref[pl.ds(b, 1), :] = (den_ref[pl.ds(b, 1), :]
                                   * jnp.exp(old_max - new_max)
                                   + tsum * jnp.exp(tmax - new_max))
        max_ref[pl.ds(b, 1), :] = new_max


def _out_body(x_ref, w_ref, bias_ref, basis_ref, max_ref, den_ref,
              out_ref, found_ref):
    r = pl.program_id(0)
    b = pl.program_id(1)
    base = r * _RT
    cout = basis_ref.shape[1]
    num_caps = x_ref.shape[2]

    x = x_ref[0, b, pl.ds(base, _RT)]
    feat = jnp.dot(x, w_ref[...], preferred_element_type=jnp.float32)
    feat = feat + bias_ref[...]
    m = feat * basis_ref[...]

    mx = max_ref[pl.ds(b, 1), :]
    den = den_ref[pl.ds(b, 1), :]
    a = jnp.exp(m - mx) / den
    amax = 1.0 / den

    tie = a == amax
    tie_f = tie.astype(jnp.float32)
    rows = jax.lax.broadcasted_iota(jnp.int32, m.shape, 0)
    first = jnp.argmax(tie_f, axis=0).astype(jnp.int32).reshape(1, -1)
    have = jnp.max(tie_f, axis=0, keepdims=True)

    prev = jnp.where(r > 0, found_ref[pl.ds(b, 1), :], 0)
    own = (have > 0.0) & (prev == 0)
    found_ref[pl.ds(b, 1), :] = prev | have.astype(jnp.int32)
    sel = tie & (rows == first) & own

    sel_paired = jnp.concatenate(
        [sel[:, :_DEPTH], sel[:, :_DEPTH], sel[:, 2 * _DEPTH:]], axis=1)

    av = a * feat
    cols = jax.lax.broadcasted_iota(jnp.int32, m.shape, 1)
    out_ref[0] = jnp.where(cols >= 2 * _DEPTH, av * (1.0 / num_caps),
                           jnp.where(sel_paired, av, 0.0))


def kernel(feat_list, W, b, caps_basis):
    L, Bv, Nv = feat_list.shape[0], feat_list.shape[1], feat_list.shape[2]
    cin = feat_list.shape[-1]
    num_caps = caps_basis.shape[1]
    cout = caps_basis.shape[3]
    xs = feat_list.reshape(L, Bv, Nv * Nv, cin)
    slab = L - 2
    basis = caps_basis.reshape(num_caps, cout)
    bias2 = b.reshape(1, cout)
    n_r = num_caps // _RT
    f32 = jnp.float32

    attn_map, mx, den = pl.pallas_call(
        _stats_body,
        grid=(n_r, Bv),
        in_specs=[
            pl.BlockSpec((1, Bv, num_caps, cin),
                         lambda r, bb: (slab, 0, 0, 0)),
            pl.BlockSpec((cin, cout), lambda r, bb: (0, 0)),
            pl.BlockSpec((1, cout), lambda r, bb: (0, 0)),
            pl.BlockSpec((_RT, cout), lambda r, bb: (r, 0)),
        ],
        out_specs=(
            pl.BlockSpec((1, _RT, cout), lambda r, bb: (bb, r, 0)),
            pl.BlockSpec((Bv, cout), lambda r, bb: (0, 0)),
            pl.BlockSpec((Bv, cout), lambda r, bb: (0, 0)),
        ),
        out_shape=(
            jax.ShapeDtypeStruct((Bv, num_caps, cout), f32),
            jax.ShapeDtypeStruct((Bv, cout), f32),
            jax.ShapeDtypeStruct((Bv, cout), f32),
        ),
        compiler_params=pltpu.CompilerParams(
            dimension_semantics=("arbitrary", "arbitrary"),
        ),
    )(xs, W, bias2, basis)

    out = pl.pallas_call(
        _out_body,
        grid=(n_r, Bv),
        in_specs=[
            pl.BlockSpec((1, Bv, num_caps, cin),
                         lambda r, bb: (slab, 0, 0, 0)),
            pl.BlockSpec((cin, cout), lambda r, bb: (0, 0)),
            pl.BlockSpec((1, cout), lambda r, bb: (0, 0)),
            pl.BlockSpec((_RT, cout), lambda r, bb: (r, 0)),
            pl.BlockSpec((Bv, cout), lambda r, bb: (0, 0)),
            pl.BlockSpec((Bv, cout), lambda r, bb: (0, 0)),
        ],
        out_specs=pl.BlockSpec((1, _RT, cout), lambda r, bb: (bb, r, 0)),
        out_shape=jax.ShapeDtypeStruct((Bv, num_caps, cout), f32),
        scratch_shapes=[pltpu.VMEM((Bv, cout), jnp.int32)],
        compiler_params=pltpu.CompilerParams(
            dimension_semantics=("arbitrary", "arbitrary"),
        ),
    )(xs, W, bias2, basis, mx, den)
    return (out, attn_map)
